# Initial kernel scaffold; baseline (speedup 1.0000x reference)
import jax
import jax.numpy as jnp
from jax.experimental import pallas as pl


def kernel(batch_emb, segment_ids, targets, W, b):
    raise NotImplementedError("write your pallas kernel here")


# speedup vs baseline: 2.4350x; 2.4350x over previous
import functools

import jax
import jax.numpy as jnp
from jax import lax
from jax.experimental import pallas as pl
from jax.experimental.pallas import tpu as pltpu
from jax.experimental.pallas import tpu_sc as plsc

N = 100000
D = 128
B = 512
T_OUT = 128
L = 16
NC = 2
NS = 16
NW = NC * NS
CHUNK = 120
SUM_BASE_LO = 3120
SUM_FULL = SUM_BASE_LO // CHUNK
CNT_BASE_LO = 6248
CNT_FULL = 6240 // CHUNK
QPT = 1024 // NW
SEG_ROWS = B // NS


def _sc_stage(batch_emb, segment_ids, t0, t1):
    mesh = plsc.VectorSubcoreMesh(core_axis_name="c", subcore_axis_name="s")

    @functools.partial(
        pl.kernel,
        mesh=mesh,
        out_type=[
            jax.ShapeDtypeStruct((NC, B, D), jnp.float32),
            jax.ShapeDtypeStruct((NC * B,), jnp.float32),
            jax.ShapeDtypeStruct((2 * B, D), jnp.float32),
        ],
        scratch_types=[
            pltpu.VMEM_SHARED((B, D), jnp.float32),
            pltpu.VMEM_SHARED((B, D), jnp.float32),
            pltpu.VMEM_SHARED((B,), jnp.int32),
            pltpu.VMEM_SHARED((B,), jnp.int32),
            pltpu.VMEM((CHUNK, D), jnp.float32),
            pltpu.VMEM((CHUNK,), jnp.int32),
            pltpu.VMEM((8,), jnp.int32),
            pltpu.VMEM((CHUNK,), jnp.int32),
            pltpu.VMEM((8,), jnp.int32),
            pltpu.VMEM((CHUNK, D), jnp.float32),
            pltpu.VMEM((SEG_ROWS, D), jnp.float32),
            pltpu.VMEM((B, D), jnp.float32),
            pltpu.VMEM((B,), jnp.int32),
            pltpu.VMEM((B,), jnp.int32),
            pltpu.VMEM((B,), jnp.float32),
            pltpu.VMEM((QPT,), jnp.int32),
            pltpu.VMEM((QPT, D), jnp.float32),
            pltpu.VMEM((QPT,), jnp.int32),
            pltpu.VMEM((QPT,), jnp.int32),
            pltpu.VMEM((QPT,), jnp.int32),
            pltpu.SemaphoreType.DMA,
        ],
    )
    def sc_kernel(emb_hbm, ids_hbm, t0_hbm, t1_hbm,
                  psums_hbm, inv_hbm, e01_hbm,
                  sums_sh, cnt_sh, starts_sh, cnts_sh,
                  rbuf, ibuf, ibuf8, cbuf_ids, cbuf8, ones_v, zrows,
                  cnt_v, starts_v, cnts_v, inv_v,
                  gidx_v, grow_v, st_s, ct_s, tg_s,
                  sem):
        cid = lax.axis_index("c")
        sid = lax.axis_index("s")
        wid = cid * NS + sid

        zv = jnp.zeros((L,), jnp.float32)

        def zfill(i, _):
            r = i // (D // L)
            k = i % (D // L)
            zrows[r, pl.ds(k * L, L)] = zv
            return 0

        lax.fori_loop(0, SEG_ROWS * (D // L), zfill, 0)

        ov = jnp.ones((L,), jnp.float32)

        def ofill(i, _):
            r = i // (D // L)
            k = i % (D // L)
            ones_v[r, pl.ds(k * L, L)] = ov
            return 0

        lax.fori_loop(0, CHUNK * (D // L), ofill, 0)

        pltpu.sync_copy(zrows, sums_sh.at[pl.ds(sid * SEG_ROWS, SEG_ROWS)])
        pltpu.sync_copy(zrows, cnt_sh.at[pl.ds(sid * SEG_ROWS, SEG_ROWS)])

        plsc.subcore_barrier()

        row_base = SUM_BASE_LO * wid + 8 * jnp.minimum(wid, 20)

        def sum_step(g, _):
            off = row_base + g * CHUNK
            pltpu.sync_copy(ids_hbm.at[pl.ds(off, CHUNK)], ibuf)
            pltpu.sync_copy(emb_hbm.at[pl.ds(off, CHUNK), :], rbuf)
            pltpu.sync_copy(rbuf, sums_sh.at[ibuf], add=True)
            return 0

        lax.fori_loop(0, SUM_FULL, sum_step, 0)

        @pl.when(wid < 20)
        def _():
            off = row_base + SUM_BASE_LO
            pltpu.sync_copy(ids_hbm.at[pl.ds(off, 8)], ibuf8)
            pltpu.sync_copy(emb_hbm.at[pl.ds(off, 8), :],
                            rbuf.at[pl.ds(0, 8), :])
            pltpu.sync_copy(rbuf.at[pl.ds(0, 8), :], sums_sh.at[ibuf8],
                            add=True)

        cnt_base = CNT_BASE_LO * sid + 8 * jnp.minimum(sid, 4)

        def cnt_step(g, _):
            off = cnt_base + g * CHUNK
            pltpu.sync_copy(ids_hbm.at[pl.ds(off, CHUNK)], cbuf_ids)
            pltpu.sync_copy(ones_v, cnt_sh.at[cbuf_ids], add=True)
            return 0

        lax.fori_loop(0, CNT_FULL, cnt_step, 0)

        def cnt_tail(off):
            pltpu.sync_copy(ids_hbm.at[pl.ds(off, 8)], cbuf8)
            pltpu.sync_copy(ones_v.at[pl.ds(0, 8), :], cnt_sh.at[cbuf8],
                            add=True)

        cnt_tail(cnt_base + CNT_FULL * CHUNK)

        @pl.when(sid < 4)
        def _():
            cnt_tail(cnt_base + CNT_FULL * CHUNK + 8)

        plsc.subcore_barrier()

        @pl.when(sid == 0)
        def _():
            pltpu.sync_copy(cnt_sh, cnt_v)
            lanes = lax.iota(jnp.int32, L)

            def cs_step(i, carry):
                zero = jnp.zeros((L,), jnp.float32)
                cvec_f = zero
                excl = carry.astype(jnp.float32)
                tot = zero
                for r in range(L):
                    c_r = cnt_v[i * L + r, pl.ds(0, L)]
                    cvec_f = jnp.where(lanes == r, c_r, cvec_f)
                    excl = excl + jnp.where(lanes > r, c_r, zero)
                    tot = tot + c_r
                cvec = cvec_f.astype(jnp.int32)
                starts_v[pl.ds(i * L, L)] = excl.astype(jnp.int32)
                cnts_v[pl.ds(i * L, L)] = cvec
                safe = jnp.maximum(cvec_f, jnp.float32(1.0))
                inv_v[pl.ds(i * L, L)] = jnp.float32(1.0) / safe
                return carry + tot.astype(jnp.int32)

            lax.fori_loop(0, B // L, cs_step, jnp.zeros((L,), jnp.int32))
            pltpu.sync_copy(starts_v, starts_sh)
            pltpu.sync_copy(cnts_v, cnts_sh)
            pltpu.sync_copy(inv_v, inv_hbm.at[pl.ds(cid * B, B)])

        plsc.subcore_barrier()

        qbase = wid * QPT
        col = qbase // B
        i0 = qbase - col * B
        pltpu.sync_copy(starts_sh.at[pl.ds(i0, QPT)], st_s)
        pltpu.sync_copy(cnts_sh.at[pl.ds(i0, QPT)], ct_s)

        @pl.when(col == 0)
        def _():
            pltpu.sync_copy(t0_hbm.at[pl.ds(i0, QPT)], tg_s)

        @pl.when(col == 1)
        def _():
            pltpu.sync_copy(t1_hbm.at[pl.ds(i0, QPT)], tg_s)

        def gi_step(k, _):
            sl = pl.ds(k * L, L)
            tv = tg_s[sl]
            cv = ct_s[sl]
            sv = st_s[sl]
            safe = jnp.maximum(cv, 1)
            loc = jnp.minimum(tv, safe - 1)
            g = sv + loc
            g = jnp.minimum(jnp.maximum(g, 0), N - 1)
            gidx_v[sl] = g
            return 0

        lax.fori_loop(0, QPT // L, gi_step, 0)
        pltpu.async_copy(emb_hbm.at[gidx_v], grow_v, sem).wait()
        pltpu.sync_copy(grow_v, e01_hbm.at[pl.ds(qbase, QPT), :])

        pltpu.sync_copy(sums_sh.at[pl.ds(sid * SEG_ROWS, SEG_ROWS)],
                        rbuf.at[pl.ds(0, SEG_ROWS), :])
        pltpu.sync_copy(rbuf.at[pl.ds(0, SEG_ROWS), :],
                        psums_hbm.at[cid, pl.ds(sid * SEG_ROWS, SEG_ROWS), :])

    return sc_kernel(batch_emb, segment_ids, t0, t1)


def _tc_body(psums_ref, invc_ref, e01_ref, w_ref, b_ref, out_ref):
    p = psums_ref[0] + psums_ref[1]
    avg = p * invc_ref[...]
    e0 = e01_ref[pl.ds(0, B), :]
    e1 = e01_ref[pl.ds(B, B), :]
    acc = jnp.dot(e0, w_ref[pl.ds(0, D), :], preferred_element_type=jnp.float32)
    acc += jnp.dot(e1, w_ref[pl.ds(D, D), :], preferred_element_type=jnp.float32)
    acc += jnp.dot(avg, w_ref[pl.ds(2 * D, D), :],
                   preferred_element_type=jnp.float32)
    out_ref[...] = acc + b_ref[...]


@jax.jit
def kernel(batch_emb, segment_ids, targets, W, b):
    t0 = targets[:, 0]
    t1 = targets[:, 1]
    psums, inv, e01 = _sc_stage(batch_emb, segment_ids, t0, t1)
    inv_col = inv[:B][:, None]
    b_row = b[None, :]
    out = pl.pallas_call(
        _tc_body,
        out_shape=jax.ShapeDtypeStruct((B, T_OUT), jnp.float32),
    )(psums, inv_col, e01, W, b_row)
    return out


# speedup vs baseline: 4.5318x; 1.8612x over previous
import functools

import jax
import jax.numpy as jnp
from jax import lax
from jax.experimental import pallas as pl
from jax.experimental.pallas import tpu as pltpu
from jax.experimental.pallas import tpu_sc as plsc

N = 100000
D = 128
B = 512
T_OUT = 128
L = 16
NC = 2
NS = 16
NW = NC * NS
CHUNK = 120
SUM_BASE_LO = 3120
SUM_FULL = SUM_BASE_LO // CHUNK
QPT = 1024 // NW
SEG_ROWS = B // NS
SRCH_ROUNDS = 17


def _sc_stage(batch_emb, segment_ids, t0, t1):
    mesh = plsc.VectorSubcoreMesh(core_axis_name="c", subcore_axis_name="s")

    @functools.partial(
        pl.kernel,
        mesh=mesh,
        out_type=[
            jax.ShapeDtypeStruct((NC, B, D), jnp.float32),
            jax.ShapeDtypeStruct((NC * B,), jnp.float32),
            jax.ShapeDtypeStruct((2 * B, D), jnp.float32),
        ],
        scratch_types=[
            pltpu.VMEM_SHARED((B, D), jnp.float32),
            pltpu.VMEM((CHUNK, D), jnp.float32),
            pltpu.VMEM((CHUNK, D), jnp.float32),
            pltpu.VMEM((CHUNK,), jnp.int32),
            pltpu.VMEM((CHUNK,), jnp.int32),
            pltpu.VMEM((8,), jnp.int32),
            pltpu.VMEM((SEG_ROWS, D), jnp.float32),
            pltpu.VMEM((4 * L,), jnp.int32),
            pltpu.VMEM((4 * L,), jnp.int32),
            pltpu.VMEM((QPT,), jnp.int32),
            pltpu.VMEM((QPT, D), jnp.float32),
            pltpu.VMEM((QPT,), jnp.int32),
            pltpu.VMEM((QPT,), jnp.float32),
            pltpu.SemaphoreType.DMA,
            pltpu.SemaphoreType.DMA,
            pltpu.SemaphoreType.DMA,
        ],
    )
    def sc_kernel(emb_hbm, ids_hbm, t0_hbm, t1_hbm,
                  psums_hbm, inv_hbm, e01_hbm,
                  sums_sh,
                  rbufA, rbufB, ibufA, ibufB, ibuf8, zrows,
                  midbuf, valbuf, gidx_v, grow_v, tg_s, inv_b,
                  semA, semB, semG):
        cid = lax.axis_index("c")
        sid = lax.axis_index("s")
        wid = cid * NS + sid

        zv = jnp.zeros((L,), jnp.float32)

        def zfill(i, _):
            r = i // (D // L)
            k = i % (D // L)
            zrows[r, pl.ds(k * L, L)] = zv
            return 0

        lax.fori_loop(0, SEG_ROWS * (D // L), zfill, 0)
        pltpu.sync_copy(zrows, sums_sh.at[pl.ds(sid * SEG_ROWS, SEG_ROWS)])
        plsc.subcore_barrier()

        row_base = SUM_BASE_LO * wid + 8 * jnp.minimum(wid, 20)

        def loads_start(rb, ib, c, sem):
            off = row_base + c * CHUNK
            pltpu.make_async_copy(ids_hbm.at[pl.ds(off, CHUNK)], ib,
                                  sem).start()
            pltpu.make_async_copy(emb_hbm.at[pl.ds(off, CHUNK), :], rb,
                                  sem).start()

        def loads_wait(rb, ib, sem):
            pltpu.make_async_copy(ids_hbm.at[pl.ds(0, CHUNK)], ib,
                                  sem).wait()
            pltpu.make_async_copy(emb_hbm.at[pl.ds(0, CHUNK), :], rb,
                                  sem).wait()

        loads_start(rbufA, ibufA, 0, semA)
        loads_start(rbufB, ibufB, 1, semB)

        def pair(g, _):
            c0 = 2 * g
            loads_wait(rbufA, ibufA, semA)
            pltpu.sync_copy(rbufA, sums_sh.at[ibufA], add=True)

            @pl.when(c0 + 2 < SUM_FULL)
            def _():
                loads_start(rbufA, ibufA, c0 + 2, semA)

            loads_wait(rbufB, ibufB, semB)
            pltpu.sync_copy(rbufB, sums_sh.at[ibufB], add=True)

            @pl.when(c0 + 3 < SUM_FULL)
            def _():
                loads_start(rbufB, ibufB, c0 + 3, semB)

            return 0

        lax.fori_loop(0, SUM_FULL // 2, pair, 0)

        @pl.when(wid < 20)
        def _():
            off = row_base + SUM_BASE_LO
            pltpu.sync_copy(ids_hbm.at[pl.ds(off, 8)], ibuf8)
            pltpu.sync_copy(emb_hbm.at[pl.ds(off, 8), :],
                            rbufA.at[pl.ds(0, 8), :])
            pltpu.sync_copy(rbufA.at[pl.ds(0, 8), :], sums_sh.at[ibuf8],
                            add=True)

        qbase = wid * QPT
        col = qbase // B
        i0 = qbase - col * B
        lanes = lax.iota(jnp.int32, L)
        tgts = [i0 + lanes, i0 + L + lanes,
                i0 + 1 + lanes, i0 + L + 1 + lanes]

        def srch(r, carry):
            los, his = carry
            for k in range(4):
                mid = lax.shift_right_arithmetic(los[k] + his[k], 1)
                midbuf[pl.ds(k * L, L)] = mid
            pltpu.async_copy(ids_hbm.at[midbuf], valbuf, semG).wait()
            nlos, nhis = [], []
            for k in range(4):
                mid = lax.shift_right_arithmetic(los[k] + his[k], 1)
                v = valbuf[pl.ds(k * L, L)]
                cond = v < tgts[k]
                nlos.append(jnp.where(cond, mid + 1, los[k]))
                nhis.append(jnp.where(cond, his[k], mid))
            return tuple(nlos), tuple(nhis)

        zero_v = jnp.zeros((L,), jnp.int32)
        n_v = jnp.full((L,), N, jnp.int32)
        los, _ = lax.fori_loop(
            0, SRCH_ROUNDS, srch,
            ((zero_v, zero_v, zero_v, zero_v), (n_v, n_v, n_v, n_v)))
        starts = [los[0], los[1]]
        cnts = [los[2] - los[0], los[3] - los[1]]

        @pl.when(col == 0)
        def _():
            pltpu.sync_copy(t0_hbm.at[pl.ds(i0, QPT)], tg_s)

        @pl.when(col == 1)
        def _():
            pltpu.sync_copy(t1_hbm.at[pl.ds(i0, QPT)], tg_s)

        for k in range(2):
            sl = pl.ds(k * L, L)
            tv = tg_s[sl]
            safe = jnp.maximum(cnts[k], 1)
            loc = jnp.minimum(tv, safe - 1)
            g = starts[k] + loc
            g = jnp.minimum(jnp.maximum(g, 0), N - 1)
            gidx_v[sl] = g
            inv_b[sl] = jnp.float32(1.0) / safe.astype(jnp.float32)

        pltpu.async_copy(emb_hbm.at[gidx_v], grow_v, semG).wait()
        pltpu.sync_copy(grow_v, e01_hbm.at[pl.ds(qbase, QPT), :])
        pltpu.sync_copy(inv_b, inv_hbm.at[pl.ds(cid * B + i0, QPT)])

        plsc.subcore_barrier()
        pltpu.sync_copy(sums_sh.at[pl.ds(sid * SEG_ROWS, SEG_ROWS)],
                        rbufA.at[pl.ds(0, SEG_ROWS), :])
        pltpu.sync_copy(rbufA.at[pl.ds(0, SEG_ROWS), :],
                        psums_hbm.at[cid, pl.ds(sid * SEG_ROWS, SEG_ROWS), :])

    return sc_kernel(batch_emb, segment_ids, t0, t1)


def _tc_body(psums_ref, invc_ref, e01_ref, w_ref, b_ref, out_ref):
    p = psums_ref[0] + psums_ref[1]
    avg = p * invc_ref[...]
    e0 = e01_ref[pl.ds(0, B), :]
    e1 = e01_ref[pl.ds(B, B), :]
    acc = jnp.dot(e0, w_ref[pl.ds(0, D), :], preferred_element_type=jnp.float32)
    acc += jnp.dot(e1, w_ref[pl.ds(D, D), :], preferred_element_type=jnp.float32)
    acc += jnp.dot(avg, w_ref[pl.ds(2 * D, D), :],
                   preferred_element_type=jnp.float32)
    out_ref[...] = acc + b_ref[...]


@jax.jit
def kernel(batch_emb, segment_ids, targets, W, b):
    t0 = targets[:, 0]
    t1 = targets[:, 1]
    psums, inv, e01 = _sc_stage(batch_emb, segment_ids, t0, t1)
    inv_col = inv[:B][:, None]
    b_row = b[None, :]
    out = pl.pallas_call(
        _tc_body,
        out_shape=jax.ShapeDtypeStruct((B, T_OUT), jnp.float32),
    )(psums, inv_col, e01, W, b_row)
    return out
